# Initial kernel scaffold; baseline (speedup 1.0000x reference)
import jax
import jax.numpy as jnp
from jax.experimental import pallas as pl


def kernel(x, edge_weight, W1, b1, Wrel, brel, Wroot, W2, b2, edge_index, batch):
    raise NotImplementedError("write your pallas kernel here")


# speedup vs baseline: 9.1161x; 9.1161x over previous
import functools
import numpy as np
import jax
import jax.numpy as jnp
from jax import lax
from jax.experimental import pallas as pl
from jax.experimental.pallas import tpu as pltpu
from jax.experimental.pallas import tpu_sc as plsc

NC = 2
NS = 16
NW = NC * NS
LANES = 128
N_NODES = 10000
E_EDGES = 320000
P_DIM = 16

NSH = 8
NCH = 320
QCH = NCH // 4
EPAD = NSH * NCH * LANES
NP_ = (-(-N_NODES // 256)) * 256
NPQ = NP_ * 4
BLK = 16
NBLK = NCH // BLK

_mesh = plsc.VectorSubcoreMesh(core_axis_name="c", subcore_axis_name="s")


def _zero_flat(ref, nelems):
    z = jnp.zeros((16,), jnp.float32)

    def body(i, _):
        ref[pl.ds(i * 16, 16)] = z
        return 0

    lax.fori_loop(0, nelems // 16, body, 0, unroll=8)


@functools.partial(
    pl.kernel,
    mesh=_mesh,
    compiler_params=pltpu.CompilerParams(needs_layout_passes=False),
    out_type=jax.ShapeDtypeStruct((NW, NP_), jnp.float32),
    scratch_types=[
        pltpu.VMEM((QCH, LANES), jnp.int32),
        pltpu.VMEM((QCH, LANES), jnp.float32),
        pltpu.VMEM((NP_,), jnp.float32),
    ],
)
def _sc_deg(col_hbm, w_hbm, out_hbm, colv, wv, accum):
    c = lax.axis_index("c")
    s = lax.axis_index("s")
    wid = s * NC + c
    gs = c * 4 + s // 4
    q = s % 4
    pltpu.sync_copy(col_hbm.at[gs, pl.ds(q * QCH, QCH)], colv)
    pltpu.sync_copy(w_hbm.at[gs, pl.ds(q * QCH, QCH)], wv)
    _zero_flat(accum, NP_)

    def chunk(j, _):
        def grp(g, _):
            cv = colv[j, pl.ds(g * 16, 16)]
            w16 = wv[j, pl.ds(g * 16, 16)]
            plsc.addupdate_scatter(accum, [cv], w16)
            return 0

        lax.fori_loop(0, LANES // 16, grp, 0)
        return 0

    lax.fori_loop(0, QCH, chunk, 0)
    pltpu.sync_copy(accum, out_hbm.at[wid])


@functools.partial(
    pl.kernel,
    mesh=_mesh,
    compiler_params=pltpu.CompilerParams(needs_layout_passes=False),
    out_type=[
        jax.ShapeDtypeStruct((NW, NP_), jnp.float32),
        jax.ShapeDtypeStruct((NSH, NCH, LANES), jnp.float32),
    ],
    scratch_types=[
        pltpu.VMEM((QCH, LANES), jnp.int32),
        pltpu.VMEM((QCH, LANES), jnp.int32),
        pltpu.VMEM((QCH, LANES), jnp.float32),
        pltpu.VMEM((QCH, LANES), jnp.float32),
        pltpu.VMEM((NP_,), jnp.float32),
        pltpu.VMEM((NP_,), jnp.float32),
    ],
)
def _sc_deg2(row_hbm, col_hbm, w_hbm, m_hbm, out_hbm, ew2_hbm,
             rowv, colv, wv, ew2v, mv, accum):
    c = lax.axis_index("c")
    s = lax.axis_index("s")
    wid = s * NC + c
    gs = c * 4 + s // 4
    q = s % 4
    pltpu.sync_copy(row_hbm.at[gs, pl.ds(q * QCH, QCH)], rowv)
    pltpu.sync_copy(col_hbm.at[gs, pl.ds(q * QCH, QCH)], colv)
    pltpu.sync_copy(w_hbm.at[gs, pl.ds(q * QCH, QCH)], wv)
    pltpu.sync_copy(m_hbm, mv)
    _zero_flat(accum, NP_)

    def chunk(j, _):
        def grp(g, _):
            rv = rowv[j, pl.ds(g * 16, 16)]
            cv = colv[j, pl.ds(g * 16, 16)]
            w16 = wv[j, pl.ds(g * 16, 16)]
            mr = plsc.load_gather(mv, [rv])
            mc = plsc.load_gather(mv, [cv])
            e2 = w16 * mr * mc
            ew2v[j, pl.ds(g * 16, 16)] = e2
            plsc.addupdate_scatter(accum, [cv], e2)
            return 0

        lax.fori_loop(0, LANES // 16, grp, 0)
        return 0

    lax.fori_loop(0, QCH, chunk, 0)
    pltpu.sync_copy(accum, out_hbm.at[wid])
    pltpu.sync_copy(ew2v, ew2_hbm.at[gs, pl.ds(q * QCH, QCH)])


@functools.partial(
    pl.kernel,
    mesh=_mesh,
    compiler_params=pltpu.CompilerParams(needs_layout_passes=False),
    out_type=jax.ShapeDtypeStruct((NW, NPQ), jnp.float32),
    scratch_types=[
        pltpu.VMEM((BLK, LANES), jnp.int32),
        pltpu.VMEM((BLK, LANES), jnp.int32),
        pltpu.VMEM((BLK, LANES), jnp.float32),
        pltpu.VMEM((NPQ,), jnp.float32),
        pltpu.VMEM((NPQ,), jnp.float32),
    ],
)
def _sc_pass(row_hbm, col_hbm, w_hbm, tabq_hbm, out_hbm,
             rowb, colb, wb, tabv, accq):
    c = lax.axis_index("c")
    s = lax.axis_index("s")
    wid = s * NC + c
    gs = c * 4 + s // 4
    fq = s % 4
    pltpu.sync_copy(tabq_hbm.at[fq], tabv)
    _zero_flat(accq, NPQ)

    def block(b, _):
        pltpu.sync_copy(row_hbm.at[gs, pl.ds(b * BLK, BLK)], rowb)
        pltpu.sync_copy(col_hbm.at[gs, pl.ds(b * BLK, BLK)], colb)
        pltpu.sync_copy(w_hbm.at[gs, pl.ds(b * BLK, BLK)], wb)

        def chunk(j, _):
            def grp(g, _):
                rv = rowb[j, pl.ds(g * 16, 16)]
                cv = colb[j, pl.ds(g * 16, 16)]
                w16 = wb[j, pl.ds(g * 16, 16)]
                rv4 = rv * 4
                cv4 = cv * 4
                for ff in range(4):
                    t = plsc.load_gather(tabv, [rv4 + ff])
                    plsc.addupdate_scatter(accq, [cv4 + ff], t * w16)
                return 0

            lax.fori_loop(0, LANES // 16, grp, 0)
            return 0

        lax.fori_loop(0, BLK, chunk, 0)
        return 0

    lax.fori_loop(0, NBLK, block, 0)
    pltpu.sync_copy(accq, out_hbm.at[wid])


GR = NP_ // 8


def _sel8():
    i0 = lax.broadcasted_iota(jnp.int32, (8, 128), 0)
    i1 = lax.broadcasted_iota(jnp.int32, (8, 128), 1)
    return (i0 == i1 // 16).astype(jnp.float32)


def _sel8t():
    i0 = lax.broadcasted_iota(jnp.int32, (128, 8), 0)
    i1 = lax.broadcasted_iota(jnp.int32, (128, 8), 1)
    return (i0 // 16 == i1).astype(jnp.float32)


def _hp_dot(a, b):
    return jnp.dot(a, b, precision=lax.Precision.HIGHEST,
                   preferred_element_type=jnp.float32)


def _tc1_body(xg_ref, w1g_ref, degp_ref, xw_ref, u_ref, d1b_ref):
    xw = _hp_dot(xg_ref[...], w1g_ref[...])
    d8 = lax.rsqrt(1.0 + jnp.sum(degp_ref[...], axis=0))
    d1b = _hp_dot(d8, _sel8())
    xw_ref[...] = xw
    u_ref[...] = xw * d1b
    d1b_ref[...] = d1b


def _tc2_body(sp_ref, d1b_ref, xw_ref, b1t_ref, h1_ref):
    d1b = d1b_ref[...]
    pre = d1b * jnp.sum(sp_ref[...], axis=0) \
        + d1b * d1b * xw_ref[...] + b1t_ref[...]
    h1_ref[...] = jnp.maximum(pre, 0.0)


def _tc3_body(aggp_ref, h1_ref, wrelt_ref, wroott_ref, brel_ref,
              m8_ref, xp_ref, *, k):
    agg = jnp.sum(aggp_ref[...], axis=0)
    h1 = h1_ref[...]
    z8 = _hp_dot(agg * wrelt_ref[...] + h1 * wroott_ref[...],
                 _sel8t()) + brel_ref[0, 0]
    score8 = jnp.tanh(z8)
    bits = lax.bitcast_convert_type(score8, jnp.int32)
    ukey = jnp.where(bits < 0, ~bits,
                     bits ^ jnp.int32(-2147483648)).astype(jnp.uint32)
    nid = lax.broadcasted_iota(jnp.int32, (GR, 8), 0) * 8 \
        + lax.broadcasted_iota(jnp.int32, (GR, 8), 1)
    ukey = jnp.where(nid < N_NODES, ukey, jnp.uint32(0))

    def bis(i, lohi):
        lo, hi = lohi
        mid = lo + (hi - lo) // 2 + jnp.uint32(1)
        cnt = jnp.sum((ukey >= mid).astype(jnp.int32))
        ge = cnt >= k
        return jnp.where(ge, mid, lo), jnp.where(ge, hi, mid - jnp.uint32(1))

    lo, _ = lax.fori_loop(0, 33, bis,
                          (jnp.uint32(0), jnp.uint32(0xFFFFFFFF)))
    m8 = (ukey >= lo).astype(jnp.float32)
    m8_ref[...] = m8
    sm128 = _hp_dot(score8 * m8, _sel8())
    xp_ref[...] = h1 * sm128


def _tc4_body(deg2p_ref, xp_ref, d2b_ref, v_ref):
    d8 = lax.rsqrt(1.0 + jnp.sum(deg2p_ref[...], axis=0))
    d2b = _hp_dot(d8, _sel8())
    d2b_ref[...] = d2b
    v_ref[...] = d2b * xp_ref[...]


def _tc5_body(s2p_ref, d2b_ref, xp_ref, w2g_ref, b2t_ref, m8_ref, out_ref,
              *, k):
    d2b = d2b_ref[...]
    pre = d2b * jnp.sum(s2p_ref[...], axis=0) + d2b * d2b * xp_ref[...]
    h2 = _hp_dot(pre, w2g_ref[...]) + b2t_ref[...]
    i0 = lax.broadcasted_iota(jnp.int32, (8, 1024), 0)
    i1 = lax.broadcasted_iota(jnp.int32, (8, 1024), 1)
    msel = (i0 == i1 // 128).astype(jnp.float32)
    h2 = jnp.maximum(h2, 0.0) * _hp_dot(m8_ref[...], msel)
    colsum = _hp_dot(jnp.ones((1, GR), jnp.float32), h2)
    f0 = lax.broadcasted_iota(jnp.int32, (1024, 128), 0)
    f1 = lax.broadcasted_iota(jnp.int32, (1024, 128), 1)
    fold = (f0 % 128 == f1).astype(jnp.float32)
    out_ref[...] = _hp_dot(colsum, fold) / float(k)


def _tc_call(body, out_shapes):
    return pl.pallas_call(body, out_shape=out_shapes)


def _quarters(tab_g):
    return tab_g.reshape(NP_, 4, 4).transpose(1, 0, 2).reshape(4, NPQ)


def _combine_layout(parts):
    p = parts.reshape(4, 4, NC, NP_, 4)
    p = p.transpose(0, 2, 3, 1, 4)
    return p.reshape(NSH, GR, 128)


@jax.jit
def _pipeline(x, edge_weight, W1, b1, Wrel, brel, Wroot, W2, b2, edge_index):
    N, F = x.shape
    H = W2.shape[1]
    k = int(np.ceil(0.8 * N_NODES))
    f32 = jnp.float32

    row = edge_index[0]
    col = edge_index[1]
    pad = EPAD - E_EDGES
    pad_idx = (jnp.arange(pad, dtype=jnp.int32) * 97) % N_NODES
    row_sh = jnp.concatenate([row, pad_idx]).reshape(NSH, NCH, LANES)
    col_sh = jnp.concatenate([col, pad_idx]).reshape(NSH, NCH, LANES)
    ew_sh = jnp.concatenate(
        [edge_weight, jnp.zeros((pad,), f32)]).reshape(NSH, NCH, LANES)
    xg = jnp.pad(x, ((0, NP_ - N), (0, 0))).reshape(GR, 8 * F)
    eye8 = jnp.eye(8, dtype=f32)
    w1g = jnp.kron(eye8, W1)
    w2g = jnp.kron(eye8, W2)
    b1t = jnp.tile(b1, 8)[None, :]
    b2t = jnp.tile(b2, 8)[None, :]
    wrelt = jnp.tile(Wrel[:, 0], 8)[None, :]
    wroott = jnp.tile(Wroot[:, 0], 8)[None, :]

    degp = _sc_deg(col_sh, ew_sh).reshape(NW, GR, 8)
    xw1, u, d1b = _tc_call(_tc1_body, [
        jax.ShapeDtypeStruct((GR, 128), f32),
        jax.ShapeDtypeStruct((GR, 128), f32),
        jax.ShapeDtypeStruct((GR, 128), f32),
    ])(xg, w1g, degp)

    sp = _combine_layout(_sc_pass(row_sh, col_sh, ew_sh, _quarters(u)))
    h1 = _tc_call(_tc2_body, jax.ShapeDtypeStruct((GR, 128), f32))(
        sp, d1b, xw1, b1t)

    aggp = _combine_layout(_sc_pass(row_sh, col_sh, ew_sh, _quarters(h1)))
    m8, xp = _tc_call(functools.partial(_tc3_body, k=k), [
        jax.ShapeDtypeStruct((GR, 8), f32),
        jax.ShapeDtypeStruct((GR, 128), f32),
    ])(aggp, h1, wrelt, wroott, brel[None, :])

    deg2p, ew2_sh = _sc_deg2(row_sh, col_sh, ew_sh, m8.reshape(NP_))
    deg2p = deg2p.reshape(NW, GR, 8)
    d2b, v = _tc_call(_tc4_body, [
        jax.ShapeDtypeStruct((GR, 128), f32),
        jax.ShapeDtypeStruct((GR, 128), f32),
    ])(deg2p, xp)

    s2p = _combine_layout(_sc_pass(row_sh, col_sh, ew2_sh, _quarters(v)))
    out = _tc_call(functools.partial(_tc5_body, k=k),
                   jax.ShapeDtypeStruct((1, H), f32))(
        s2p, d2b, xp, w2g, b2t, m8)
    return out


def kernel(x, edge_weight, W1, b1, Wrel, brel, Wroot, W2, b2, edge_index,
           batch):
    del batch
    return _pipeline(x, edge_weight, W1, b1, Wrel, brel, Wroot, W2, b2,
                     edge_index)


# speedup vs baseline: 47.5650x; 5.2177x over previous
import functools
import numpy as np
import jax
import jax.numpy as jnp
from jax import lax
from jax.experimental import pallas as pl
from jax.experimental.pallas import tpu as pltpu
from jax.experimental.pallas import tpu_sc as plsc

NC = 2
NS = 16
NW = NC * NS
LANES = 128
N_NODES = 10000
E_EDGES = 320000
P_DIM = 16

NSH = 8
NCH = 320
QCH = NCH // 4
EPAD = NSH * NCH * LANES
NP_ = (-(-N_NODES // 256)) * 256
NPQ = NP_ * 4
BLK = 16
NBLK = NCH // BLK

_mesh = plsc.VectorSubcoreMesh(core_axis_name="c", subcore_axis_name="s")


def _zero_flat(ref, nelems):
    z = jnp.zeros((16,), jnp.float32)

    def body(i, _):
        ref[pl.ds(i * 16, 16)] = z
        return 0

    lax.fori_loop(0, nelems // 16, body, 0, unroll=8)


@functools.partial(
    pl.kernel,
    mesh=_mesh,
    compiler_params=pltpu.CompilerParams(needs_layout_passes=False),
    out_type=jax.ShapeDtypeStruct((NW, NP_), jnp.float32),
    scratch_types=[
        pltpu.VMEM((QCH, LANES), jnp.int32),
        pltpu.VMEM((QCH, LANES), jnp.float32),
        pltpu.VMEM((NP_,), jnp.float32),
    ],
)
def _sc_deg(col_hbm, w_hbm, out_hbm, colv, wv, accum):
    c = lax.axis_index("c")
    s = lax.axis_index("s")
    wid = s * NC + c
    gs = c * 4 + s // 4
    q = s % 4
    pltpu.sync_copy(col_hbm.at[gs, pl.ds(q * QCH, QCH)], colv)
    pltpu.sync_copy(w_hbm.at[gs, pl.ds(q * QCH, QCH)], wv)
    _zero_flat(accum, NP_)

    def chunk(j, _):
        def grp(g, _):
            cv = colv[j, pl.ds(g * 16, 16)]
            w16 = wv[j, pl.ds(g * 16, 16)]
            plsc.addupdate_scatter(accum, [cv], w16)
            return 0

        lax.fori_loop(0, LANES // 16, grp, 0)
        return 0

    lax.fori_loop(0, QCH, chunk, 0)
    pltpu.sync_copy(accum, out_hbm.at[wid])


@functools.partial(
    pl.kernel,
    mesh=_mesh,
    compiler_params=pltpu.CompilerParams(needs_layout_passes=False),
    out_type=[
        jax.ShapeDtypeStruct((NW, NP_), jnp.float32),
        jax.ShapeDtypeStruct((NSH, NCH, LANES), jnp.float32),
    ],
    scratch_types=[
        pltpu.VMEM((QCH, LANES), jnp.int32),
        pltpu.VMEM((QCH, LANES), jnp.int32),
        pltpu.VMEM((QCH, LANES), jnp.float32),
        pltpu.VMEM((QCH, LANES), jnp.float32),
        pltpu.VMEM((NP_,), jnp.float32),
        pltpu.VMEM((NP_,), jnp.float32),
    ],
)
def _sc_deg2(row_hbm, col_hbm, w_hbm, m_hbm, out_hbm, ew2_hbm,
             rowv, colv, wv, ew2v, mv, accum):
    c = lax.axis_index("c")
    s = lax.axis_index("s")
    wid = s * NC + c
    gs = c * 4 + s // 4
    q = s % 4
    pltpu.sync_copy(row_hbm.at[gs, pl.ds(q * QCH, QCH)], rowv)
    pltpu.sync_copy(col_hbm.at[gs, pl.ds(q * QCH, QCH)], colv)
    pltpu.sync_copy(w_hbm.at[gs, pl.ds(q * QCH, QCH)], wv)
    pltpu.sync_copy(m_hbm, mv)
    _zero_flat(accum, NP_)

    def chunk(j, _):
        def grp(g, _):
            rv = rowv[j, pl.ds(g * 16, 16)]
            cv = colv[j, pl.ds(g * 16, 16)]
            w16 = wv[j, pl.ds(g * 16, 16)]
            mr = plsc.load_gather(mv, [rv])
            mc = plsc.load_gather(mv, [cv])
            e2 = w16 * mr * mc
            ew2v[j, pl.ds(g * 16, 16)] = e2
            plsc.addupdate_scatter(accum, [cv], e2)
            return 0

        lax.fori_loop(0, LANES // 16, grp, 0)
        return 0

    lax.fori_loop(0, QCH, chunk, 0)
    pltpu.sync_copy(accum, out_hbm.at[wid])
    pltpu.sync_copy(ew2v, ew2_hbm.at[gs, pl.ds(q * QCH, QCH)])


@functools.partial(
    pl.kernel,
    mesh=_mesh,
    compiler_params=pltpu.CompilerParams(needs_layout_passes=False),
    out_type=jax.ShapeDtypeStruct((NW, 4, NP_), jnp.float32),
    scratch_types=[
        pltpu.VMEM((BLK, LANES), jnp.int32),
        pltpu.VMEM((BLK, LANES), jnp.int32),
        pltpu.VMEM((BLK, LANES), jnp.float32),
        pltpu.VMEM((4, NP_), jnp.float32),
        pltpu.VMEM((4, NP_), jnp.float32),
    ],
)
def _sc_pass(row_hbm, col_hbm, w_hbm, tabt_hbm, out_hbm,
             rowb, colb, wb, tabv, accq):
    c = lax.axis_index("c")
    s = lax.axis_index("s")
    wid = s * NC + c
    gs = c * 4 + s // 4
    fq = s % 4
    pltpu.sync_copy(tabt_hbm.at[pl.ds(fq * 4, 4)], tabv)
    z = jnp.zeros((16,), jnp.float32)

    def zrow(i, _):
        for ff in range(4):
            accq[ff, pl.ds(i * 16, 16)] = z
        return 0

    lax.fori_loop(0, NP_ // 16, zrow, 0, unroll=8)
    ffs = [jnp.full((16,), ff, jnp.int32) for ff in range(4)]

    def block(b, _):
        pltpu.sync_copy(row_hbm.at[gs, pl.ds(b * BLK, BLK)], rowb)
        pltpu.sync_copy(col_hbm.at[gs, pl.ds(b * BLK, BLK)], colb)
        pltpu.sync_copy(w_hbm.at[gs, pl.ds(b * BLK, BLK)], wb)

        def chunk(j, _):
            def grp(g, _):
                rv = rowb[j, pl.ds(g * 16, 16)]
                cv = colb[j, pl.ds(g * 16, 16)]
                w16 = wb[j, pl.ds(g * 16, 16)]
                for ff in range(4):
                    t = plsc.load_gather(tabv, [ffs[ff], rv])
                    plsc.addupdate_scatter(accq, [ffs[ff], cv], t * w16)
                return 0

            lax.fori_loop(0, LANES // 16, grp, 0)
            return 0

        lax.fori_loop(0, BLK, chunk, 0)
        return 0

    lax.fori_loop(0, NBLK, block, 0)
    pltpu.sync_copy(accq, out_hbm.at[wid])


def _hp_dot(a, b):
    return jnp.dot(a, b, precision=lax.Precision.HIGHEST,
                   preferred_element_type=jnp.float32)


def _tc1_body(xt_ref, w1t_ref, degp_ref, xw_ref, u_ref, d1_ref):
    xw = _hp_dot(w1t_ref[...], xt_ref[...])
    d1 = lax.rsqrt(1.0 + jnp.sum(degp_ref[...], axis=0))
    xw_ref[...] = xw
    u_ref[...] = xw * d1[None, :]
    d1_ref[...] = d1


def _tc2_body(c0, c1, c2, c3, d1_ref, xw_ref, b1c_ref, h1_ref):
    d1 = d1_ref[...]
    d1sq = d1 * d1
    for fq, cref in enumerate((c0, c1, c2, c3)):
        sq = jnp.sum(cref[...], axis=0)
        sl = pl.ds(fq * 4, 4)
        pre = d1[None, :] * sq + d1sq[None, :] * xw_ref[sl, :] \
            + b1c_ref[sl, :]
        h1_ref[sl, :] = jnp.maximum(pre, 0.0)


def _tc3_body(a0, a1, a2, a3, h1_ref, wrel_ref, wroot_ref, brel_ref,
              m_ref, xp_ref, *, k):
    h1 = h1_ref[...]
    z = jnp.sum(h1 * wroot_ref[...], axis=0) + brel_ref[0, 0]
    for fq, aref in enumerate((a0, a1, a2, a3)):
        agg = jnp.sum(aref[...], axis=0)
        z = z + jnp.sum(agg * wrel_ref[pl.ds(fq * 4, 4), :], axis=0)
    score = jnp.tanh(z)
    bits = lax.bitcast_convert_type(score, jnp.int32)
    ukey = jnp.where(bits < 0, ~bits,
                     bits ^ jnp.int32(-2147483648)).astype(jnp.uint32)
    nid = lax.broadcasted_iota(jnp.int32, (NP_,), 0)
    ukey = jnp.where(nid < N_NODES, ukey, jnp.uint32(0))

    def bis(i, lohi):
        lo, hi = lohi
        mid = lo + (hi - lo) // 2 + jnp.uint32(1)
        cnt = jnp.sum((ukey >= mid).astype(jnp.int32))
        ge = cnt >= k
        return jnp.where(ge, mid, lo), jnp.where(ge, hi, mid - jnp.uint32(1))

    lo, _ = lax.fori_loop(0, 33, bis,
                          (jnp.uint32(0), jnp.uint32(0xFFFFFFFF)))
    m = (ukey >= lo).astype(jnp.float32)
    m_ref[...] = m
    xp_ref[...] = h1 * (score * m)[None, :]


def _tc4_body(deg2p_ref, xp_ref, d2_ref, v_ref):
    d2 = lax.rsqrt(1.0 + jnp.sum(deg2p_ref[...], axis=0))
    d2_ref[...] = d2
    v_ref[...] = d2[None, :] * xp_ref[...]


def _tc5_body(s0, s1, s2, s3, d2_ref, xp_ref, w2t_ref, b2c_ref, m_ref,
              out_ref, *, k):
    d2 = d2_ref[...]
    d2sq = d2 * d2
    pres = []
    for fq, sref in enumerate((s0, s1, s2, s3)):
        sq = jnp.sum(sref[...], axis=0)
        pres.append(d2[None, :] * sq
                    + d2sq[None, :] * xp_ref[pl.ds(fq * 4, 4), :])
    pre = jnp.concatenate(pres, axis=0)
    h2 = _hp_dot(w2t_ref[...], pre) + b2c_ref[...]
    h2 = jnp.maximum(h2, 0.0) * m_ref[...][None, :]
    out_ref[...] = jnp.sum(h2, axis=1)[:, None] / float(k)


def _tc_call(body, out_shapes):
    return pl.pallas_call(body, out_shape=out_shapes)


def _chunks(parts):
    p = parts.reshape(4, 4, NC, 4, NP_)
    return [p[:, fq].reshape(NSH, 4, NP_) for fq in range(4)]


@jax.jit
def _pipeline(x, edge_weight, W1, b1, Wrel, brel, Wroot, W2, b2, edge_index):
    N, F = x.shape
    H = W2.shape[1]
    k = int(np.ceil(0.8 * N_NODES))
    f32 = jnp.float32

    row = edge_index[0]
    col = edge_index[1]
    pad = EPAD - E_EDGES
    pad_idx = (jnp.arange(pad, dtype=jnp.int32) * 97) % N_NODES
    row_sh = jnp.concatenate([row, pad_idx]).reshape(NSH, NCH, LANES)
    col_sh = jnp.concatenate([col, pad_idx]).reshape(NSH, NCH, LANES)
    ew_sh = jnp.concatenate(
        [edge_weight, jnp.zeros((pad,), f32)]).reshape(NSH, NCH, LANES)
    xt = jnp.pad(x, ((0, NP_ - N), (0, 0))).T
    w1t = W1.T
    w2t = W2.T
    b1c = b1[:, None]
    b2c = b2[:, None]

    degp = _sc_deg(col_sh, ew_sh)
    xw1, u, d1 = _tc_call(_tc1_body, [
        jax.ShapeDtypeStruct((P_DIM, NP_), f32),
        jax.ShapeDtypeStruct((P_DIM, NP_), f32),
        jax.ShapeDtypeStruct((NP_,), f32),
    ])(xt, w1t, degp)

    sp = _chunks(_sc_pass(row_sh, col_sh, ew_sh, u))
    h1 = _tc_call(_tc2_body, jax.ShapeDtypeStruct((P_DIM, NP_), f32))(
        *sp, d1, xw1, b1c)

    aggp = _chunks(_sc_pass(row_sh, col_sh, ew_sh, h1))
    m, xp = _tc_call(functools.partial(_tc3_body, k=k), [
        jax.ShapeDtypeStruct((NP_,), f32),
        jax.ShapeDtypeStruct((P_DIM, NP_), f32),
    ])(*aggp, h1, Wrel, Wroot, brel[None, :])

    deg2p, ew2_sh = _sc_deg2(row_sh, col_sh, ew_sh, m)
    d2, v = _tc_call(_tc4_body, [
        jax.ShapeDtypeStruct((NP_,), f32),
        jax.ShapeDtypeStruct((P_DIM, NP_), f32),
    ])(deg2p, xp)

    s2p = _chunks(_sc_pass(row_sh, col_sh, ew2_sh, v))
    out = _tc_call(functools.partial(_tc5_body, k=k),
                   jax.ShapeDtypeStruct((H, 1), f32))(
        *s2p, d2, xp, w2t, b2c, m)
    return out.reshape(1, H)


def kernel(x, edge_weight, W1, b1, Wrel, brel, Wroot, W2, b2, edge_index,
           batch):
    del batch
    return _pipeline(x, edge_weight, W1, b1, Wrel, brel, Wroot, W2, b2,
                     edge_index)


# speedup vs baseline: 47.8715x; 1.0064x over previous
import functools
import numpy as np
import jax
import jax.numpy as jnp
from jax import lax
from jax.experimental import pallas as pl
from jax.experimental.pallas import tpu as pltpu
from jax.experimental.pallas import tpu_sc as plsc

NC = 2
NS = 16
NW = NC * NS
LANES = 128
N_NODES = 10000
E_EDGES = 320000
P_DIM = 16

NSH = 8
NCH = 320
QCH = NCH // 4
EPAD = NSH * NCH * LANES
NP_ = (-(-N_NODES // 256)) * 256
NPQ = NP_ * 4
BLK = 16
NBLK = NCH // BLK

_mesh = plsc.VectorSubcoreMesh(core_axis_name="c", subcore_axis_name="s")


def _zero_flat(ref, nelems):
    z = jnp.zeros((16,), jnp.float32)

    def body(i, _):
        ref[pl.ds(i * 16, 16)] = z
        return 0

    lax.fori_loop(0, nelems // 16, body, 0, unroll=8)


@functools.partial(
    pl.kernel,
    mesh=_mesh,
    compiler_params=pltpu.CompilerParams(needs_layout_passes=False),
    out_type=jax.ShapeDtypeStruct((NW, NP_), jnp.float32),
    scratch_types=[
        pltpu.VMEM((QCH, LANES), jnp.int32),
        pltpu.VMEM((QCH, LANES), jnp.float32),
        pltpu.VMEM((NP_,), jnp.float32),
    ],
)
def _sc_deg(col_hbm, w_hbm, out_hbm, colv, wv, accum):
    c = lax.axis_index("c")
    s = lax.axis_index("s")
    wid = s * NC + c
    gs = c * 4 + s // 4
    q = s % 4
    pltpu.sync_copy(col_hbm.at[gs, pl.ds(q * QCH, QCH)], colv)
    pltpu.sync_copy(w_hbm.at[gs, pl.ds(q * QCH, QCH)], wv)
    _zero_flat(accum, NP_)

    def chunk(j, _):
        for g in range(LANES // 16):
            cv = colv[j, pl.ds(g * 16, 16)]
            w16 = wv[j, pl.ds(g * 16, 16)]
            plsc.addupdate_scatter(accum, [cv], w16)
        return 0

    lax.fori_loop(0, QCH, chunk, 0)
    pltpu.sync_copy(accum, out_hbm.at[wid])


@functools.partial(
    pl.kernel,
    mesh=_mesh,
    compiler_params=pltpu.CompilerParams(needs_layout_passes=False),
    out_type=[
        jax.ShapeDtypeStruct((NW, NP_), jnp.float32),
        jax.ShapeDtypeStruct((NSH, NCH, LANES), jnp.float32),
    ],
    scratch_types=[
        pltpu.VMEM((QCH, LANES), jnp.int32),
        pltpu.VMEM((QCH, LANES), jnp.int32),
        pltpu.VMEM((QCH, LANES), jnp.float32),
        pltpu.VMEM((QCH, LANES), jnp.float32),
        pltpu.VMEM((NP_,), jnp.float32),
        pltpu.VMEM((NP_,), jnp.float32),
    ],
)
def _sc_deg2(row_hbm, col_hbm, w_hbm, m_hbm, out_hbm, ew2_hbm,
             rowv, colv, wv, ew2v, mv, accum):
    c = lax.axis_index("c")
    s = lax.axis_index("s")
    wid = s * NC + c
    gs = c * 4 + s // 4
    q = s % 4
    pltpu.sync_copy(row_hbm.at[gs, pl.ds(q * QCH, QCH)], rowv)
    pltpu.sync_copy(col_hbm.at[gs, pl.ds(q * QCH, QCH)], colv)
    pltpu.sync_copy(w_hbm.at[gs, pl.ds(q * QCH, QCH)], wv)
    pltpu.sync_copy(m_hbm, mv)
    _zero_flat(accum, NP_)

    def chunk(j, _):
        for g in range(LANES // 16):
            rv = rowv[j, pl.ds(g * 16, 16)]
            cv = colv[j, pl.ds(g * 16, 16)]
            w16 = wv[j, pl.ds(g * 16, 16)]
            mr = plsc.load_gather(mv, [rv])
            mc = plsc.load_gather(mv, [cv])
            e2 = w16 * mr * mc
            ew2v[j, pl.ds(g * 16, 16)] = e2
            plsc.addupdate_scatter(accum, [cv], e2)
        return 0

    lax.fori_loop(0, QCH, chunk, 0)
    pltpu.sync_copy(accum, out_hbm.at[wid])
    pltpu.sync_copy(ew2v, ew2_hbm.at[gs, pl.ds(q * QCH, QCH)])


@functools.partial(
    pl.kernel,
    mesh=_mesh,
    compiler_params=pltpu.CompilerParams(needs_layout_passes=False),
    out_type=jax.ShapeDtypeStruct((NW, 4, NP_), jnp.float32),
    scratch_types=[
        pltpu.VMEM((BLK, LANES), jnp.int32),
        pltpu.VMEM((BLK, LANES), jnp.int32),
        pltpu.VMEM((BLK, LANES), jnp.float32),
        pltpu.VMEM((4, NP_), jnp.float32),
        pltpu.VMEM((4, NP_), jnp.float32),
    ],
)
def _sc_pass(row_hbm, col_hbm, w_hbm, tabt_hbm, out_hbm,
             rowb, colb, wb, tabv, accq):
    c = lax.axis_index("c")
    s = lax.axis_index("s")
    wid = s * NC + c
    gs = c * 4 + s // 4
    fq = s % 4
    pltpu.sync_copy(tabt_hbm.at[pl.ds(fq * 4, 4)], tabv)
    z = jnp.zeros((16,), jnp.float32)

    def zrow(i, _):
        for ff in range(4):
            accq[ff, pl.ds(i * 16, 16)] = z
        return 0

    lax.fori_loop(0, NP_ // 16, zrow, 0, unroll=8)
    ffs = [jnp.full((16,), ff, jnp.int32) for ff in range(4)]

    def block(b, _):
        pltpu.sync_copy(row_hbm.at[gs, pl.ds(b * BLK, BLK)], rowb)
        pltpu.sync_copy(col_hbm.at[gs, pl.ds(b * BLK, BLK)], colb)
        pltpu.sync_copy(w_hbm.at[gs, pl.ds(b * BLK, BLK)], wb)

        def chunk(j, _):
            for g in range(LANES // 16):
                rv = rowb[j, pl.ds(g * 16, 16)]
                cv = colb[j, pl.ds(g * 16, 16)]
                w16 = wb[j, pl.ds(g * 16, 16)]
                for ff in range(4):
                    t = plsc.load_gather(tabv, [ffs[ff], rv])
                    plsc.addupdate_scatter(accq, [ffs[ff], cv], t * w16)
            return 0

        lax.fori_loop(0, BLK, chunk, 0)
        return 0

    lax.fori_loop(0, NBLK, block, 0)
    pltpu.sync_copy(accq, out_hbm.at[wid])


def _hp_dot(a, b):
    return jnp.dot(a, b, precision=lax.Precision.HIGHEST,
                   preferred_element_type=jnp.float32)


def _bf(a):
    return a.astype(jnp.bfloat16).astype(jnp.float32)


def _tc1_body(xt_ref, w1t_ref, degp_ref, xw_ref, u_ref, d1_ref):
    xw = jnp.dot(w1t_ref[...], xt_ref[...],
                 preferred_element_type=jnp.float32)
    d1 = lax.rsqrt(1.0 + jnp.sum(degp_ref[...], axis=0))
    xw_ref[...] = xw
    u_ref[...] = xw * d1[None, :]
    d1_ref[...] = d1


def _tc2_body(c0, c1, c2, c3, d1_ref, xw_ref, b1c_ref, h1_ref):
    d1 = d1_ref[...]
    d1sq = d1 * d1
    for fq, cref in enumerate((c0, c1, c2, c3)):
        sq = jnp.sum(cref[...], axis=0)
        sl = pl.ds(fq * 4, 4)
        pre = d1[None, :] * sq + d1sq[None, :] * xw_ref[sl, :] \
            + b1c_ref[sl, :]
        h1_ref[sl, :] = jnp.maximum(pre, 0.0)


def _tc3_body(a0, a1, a2, a3, h1_ref, wrel_ref, wroot_ref, brel_ref,
              m_ref, xp_ref, *, k):
    h1 = h1_ref[...]
    z = jnp.sum(_bf(h1) * _bf(wroot_ref[...]), axis=0) + brel_ref[0, 0]
    for fq, aref in enumerate((a0, a1, a2, a3)):
        agg = jnp.sum(aref[...], axis=0)
        z = z + jnp.sum(_bf(agg) * _bf(wrel_ref[pl.ds(fq * 4, 4), :]),
                        axis=0)
    score = jnp.tanh(z)
    bits = lax.bitcast_convert_type(score, jnp.int32)
    ukey = jnp.where(bits < 0, ~bits,
                     bits ^ jnp.int32(-2147483648)).astype(jnp.uint32)
    nid = lax.broadcasted_iota(jnp.int32, (NP_,), 0)
    ukey = jnp.where(nid < N_NODES, ukey, jnp.uint32(0))

    def bis(i, lohi):
        lo, hi = lohi
        mid = lo + (hi - lo) // 2 + jnp.uint32(1)
        cnt = jnp.sum((ukey >= mid).astype(jnp.int32))
        ge = cnt >= k
        return jnp.where(ge, mid, lo), jnp.where(ge, hi, mid - jnp.uint32(1))

    lo, _ = lax.fori_loop(0, 33, bis,
                          (jnp.uint32(0), jnp.uint32(0xFFFFFFFF)))
    m = (ukey >= lo).astype(jnp.float32)
    m_ref[...] = m
    xp_ref[...] = h1 * (score * m)[None, :]


def _tc4_body(deg2p_ref, xp_ref, d2_ref, v_ref):
    d2 = lax.rsqrt(1.0 + jnp.sum(deg2p_ref[...], axis=0))
    d2_ref[...] = d2
    v_ref[...] = d2[None, :] * _bf(xp_ref[...])


def _tc5_body(s0, s1, s2, s3, d2_ref, xp_ref, w2t_ref, b2c_ref, m_ref,
              out_ref, *, k):
    d2 = d2_ref[...]
    d2sq = d2 * d2
    pres = []
    for fq, sref in enumerate((s0, s1, s2, s3)):
        sq = jnp.sum(sref[...], axis=0)
        pres.append(d2[None, :] * sq
                    + d2sq[None, :] * _bf(xp_ref[pl.ds(fq * 4, 4), :]))
    pre = jnp.concatenate(pres, axis=0)
    h2 = _hp_dot(_bf(w2t_ref[...]), pre) + b2c_ref[...]
    h2 = jnp.maximum(h2, 0.0) * m_ref[...][None, :]
    out_ref[...] = jnp.sum(h2, axis=1)[:, None] / float(k)


def _tc_call(body, out_shapes):
    return pl.pallas_call(body, out_shape=out_shapes)


def _chunks(parts):
    p = parts.reshape(4, 4, NC, 4, NP_)
    return [p[:, fq].reshape(NSH, 4, NP_) for fq in range(4)]


@jax.jit
def _pipeline(x, edge_weight, W1, b1, Wrel, brel, Wroot, W2, b2, edge_index):
    N, F = x.shape
    H = W2.shape[1]
    k = int(np.ceil(0.8 * N_NODES))
    f32 = jnp.float32

    row = edge_index[0]
    col = edge_index[1]
    pad = EPAD - E_EDGES
    pad_idx = (jnp.arange(pad, dtype=jnp.int32) * 97) % N_NODES
    row_sh = jnp.concatenate([row, pad_idx]).reshape(NSH, NCH, LANES)
    col_sh = jnp.concatenate([col, pad_idx]).reshape(NSH, NCH, LANES)
    ew_sh = jnp.concatenate(
        [edge_weight, jnp.zeros((pad,), f32)]).reshape(NSH, NCH, LANES)
    xt = jnp.pad(x, ((0, NP_ - N), (0, 0))).T
    w1t = W1.T
    w2t = W2.T
    b1c = b1[:, None]
    b2c = b2[:, None]

    degp = _sc_deg(col_sh, ew_sh)
    xw1, u, d1 = _tc_call(_tc1_body, [
        jax.ShapeDtypeStruct((P_DIM, NP_), f32),
        jax.ShapeDtypeStruct((P_DIM, NP_), f32),
        jax.ShapeDtypeStruct((NP_,), f32),
    ])(xt, w1t, degp)

    sp = _chunks(_sc_pass(row_sh, col_sh, ew_sh, u))
    h1 = _tc_call(_tc2_body, jax.ShapeDtypeStruct((P_DIM, NP_), f32))(
        *sp, d1, xw1, b1c)

    aggp = _chunks(_sc_pass(row_sh, col_sh, ew_sh, h1))
    m, xp = _tc_call(functools.partial(_tc3_body, k=k), [
        jax.ShapeDtypeStruct((NP_,), f32),
        jax.ShapeDtypeStruct((P_DIM, NP_), f32),
    ])(*aggp, h1, Wrel, Wroot, brel[None, :])

    deg2p, ew2_sh = _sc_deg2(row_sh, col_sh, ew_sh, m)
    d2, v = _tc_call(_tc4_body, [
        jax.ShapeDtypeStruct((NP_,), f32),
        jax.ShapeDtypeStruct((P_DIM, NP_), f32),
    ])(deg2p, xp)

    s2p = _chunks(_sc_pass(row_sh, col_sh, ew2_sh, v))
    out = _tc_call(functools.partial(_tc5_body, k=k),
                   jax.ShapeDtypeStruct((H, 1), f32))(
        *s2p, d2, xp, w2t, b2c, m)
    return out.reshape(1, H)


def kernel(x, edge_weight, W1, b1, Wrel, brel, Wroot, W2, b2, edge_index,
           batch):
    del batch
    return _pipeline(x, edge_weight, W1, b1, Wrel, brel, Wroot, W2, b2,
                     edge_index)
